# baseline (speedup 1.0000x reference)
import jax
import jax.numpy as jnp
from jax import lax
from jax.experimental import pallas as pl
from jax.experimental.pallas import tpu as pltpu
from jax.experimental.pallas import tpu_sc as plsc

B = 128
V = 100000
HW = 50048
CW = 2560
NFULL = 19
SUBW = 1280
SPH = 40
SPAD = 48
GP3 = SUBW // 16

_info = plsc.get_sparse_core_info()
NC, NS = _info.num_cores, _info.num_subcores


def _iota16():
    return lax.iota(jnp.int32, 16)


def _get1(ref, idx):
    return plsc.load_gather(ref, [jnp.full((16,), idx, jnp.int32)])[0]


def _get2(ref, i, j):
    return plsc.load_gather(ref, [jnp.full((16,), i, jnp.int32),
                                  jnp.full((16,), j, jnp.int32)])[0]


def _set2(ref, i, j, val):
    plsc.store_scatter(ref, [jnp.full((16,), i, jnp.int32),
                             jnp.full((16,), j, jnp.int32)],
                       jnp.full((16,), val), mask=_iota16() == 0)


def _sc_body(d_hbm, t_hbm, ids_hbm, bon_hbm, u_hbm, ru_hbm, out_hbm,
             dblks, tblks, dtl, ttl, dtl32, ttl32, qd32, qt32,
             subsums, cumbuf,
             ids_v, bon_v, u_v, ru_v, xf, xi, sharedF, sharedI,
             outb, sems):
    cid = lax.axis_index("c")
    sid = lax.axis_index("s")
    gt = 8 * cid + sid // 2
    h = sid % 2
    r0 = pl.multiple_of(8 * gt, 8)
    hoff = HW * h
    iota = _iota16()

    pltpu.sync_copy(ru_hbm, ru_v)

    chunks = [(CW * k, CW, 2) for k in range(NFULL)] + [(CW * NFULL, 1280, 1)]

    def startch(k):
        off, w, _ = chunks[k]
        b = k % 2
        col = pl.multiple_of(hoff + off, 128)
        pltpu.async_copy(d_hbm.at[pl.ds(r0, 8), pl.ds(col, w)],
                         dblks[b].at[:, pl.ds(0, w)], sems[b])
        pltpu.async_copy(t_hbm.at[pl.ds(r0, 8), pl.ds(col, w)],
                         tblks[b].at[:, pl.ds(0, w)], sems[b])

    def waitch(k):
        off, w, _ = chunks[k]
        b = k % 2
        col = pl.multiple_of(hoff + off, 128)
        pltpu.make_async_copy(d_hbm.at[pl.ds(r0, 8), pl.ds(col, w)],
                              dblks[b].at[:, pl.ds(0, w)], sems[b]).wait()
        pltpu.make_async_copy(t_hbm.at[pl.ds(r0, 8), pl.ds(col, w)],
                              tblks[b].at[:, pl.ds(0, w)], sems[b]).wait()

    for i in range(8):
        plsc.store_scatter(subsums,
                           [jnp.full((16,), i, jnp.int32), SPH + iota],
                           jnp.zeros((16,), jnp.float32),
                           mask=iota < (SPAD - SPH))

    startch(0)
    for k, (off, w, nsub) in enumerate(chunks):
        if k + 1 < len(chunks):
            startch(k + 1)
        waitch(k)
        db, tb = dblks[k % 2], tblks[k % 2]

        def row_body(row, _, db=db, tb=tb, k=k):
            rowv = jnp.full((16,), row, jnp.int32)

            def sub_body(s, _2):
                def grp(g, carry):
                    colv = carry[0]
                    accs = list(carry[1:])
                    for u in range(16):
                        cu = colv + u * 16
                        dd = plsc.load_gather(db, [rowv, cu])
                        tt = plsc.load_gather(tb, [rowv, cu])
                        accs[u % 4] = accs[u % 4] + jnp.maximum(
                            tt - dd, jnp.float32(0.0))
                    return (colv + 256,) + tuple(accs)

                z = jnp.zeros((16,), jnp.float32)
                col0 = s * SUBW + iota
                a = lax.fori_loop(0, GP3 // 16, grp, (col0, z, z, z, z))
                tot = jnp.sum((a[1] + a[2]) + (a[3] + a[4]))
                plsc.store_scatter(
                    subsums,
                    [rowv, jnp.full((16,), 2 * k, jnp.int32) + s],
                    jnp.full((16,), tot), mask=iota == 0)
                return _2

            return lax.fori_loop(0, nsub, sub_body, _)

        lax.fori_loop(0, 8, row_body, jnp.int32(0))

    @pl.when(h == 0)
    def _():
        pltpu.sync_copy(d_hbm.at[pl.ds(r0, 8), pl.ds(49920, 128)], dtl)
        pltpu.sync_copy(t_hbm.at[pl.ds(r0, 8), pl.ds(49920, 128)], ttl)

    @pl.when(h == 1)
    def _():
        pltpu.sync_copy(d_hbm.at[pl.ds(r0, 8), pl.ds(99968, 32)], dtl32)
        pltpu.sync_copy(t_hbm.at[pl.ds(r0, 8), pl.ds(99968, 32)], ttl32)

    def _tail_accum(dref, tref, ngrp):
        def tail_body(row, _):
            rowv = jnp.full((16,), row, jnp.int32)

            def tgrp(g, acc):
                colv = g * 16 + iota
                dd = plsc.load_gather(dref, [rowv, colv])
                tt = plsc.load_gather(tref, [rowv, colv])
                return acc + jnp.maximum(tt - dd, jnp.float32(0.0))

            acc = lax.fori_loop(0, ngrp, tgrp,
                                jnp.zeros((16,), jnp.float32))
            plsc.store_scatter(subsums,
                               [rowv, jnp.full((16,), SPH - 1, jnp.int32)],
                               jnp.full((16,), jnp.sum(acc)),
                               mask=iota == 0)
            return _

        lax.fori_loop(0, 8, tail_body, jnp.int32(0))

    @pl.when(h == 0)
    def _():
        _tail_accum(dtl, ttl, 8)

    @pl.when(h == 1)
    def _():
        _tail_accum(dtl32, ttl32, 2)

    mytotv = jnp.zeros((16,), jnp.float32)
    for i in range(8):
        acc = jnp.zeros((16,), jnp.float32)

        def p2a(j, acc, i=i):
            return acc + subsums[i, pl.ds(j * 16, 16)]

        acc = lax.fori_loop(0, SPAD // 16, p2a, acc)
        mytotv = jnp.where(iota == i, jnp.full((16,), jnp.sum(acc)), mytotv)

    slot = pl.multiple_of(sid * 16, 8)
    pslot = pl.multiple_of((sid ^ 1) * 16, 8)
    xf[...] = mytotv
    pltpu.sync_copy(xf, sharedF.at[pl.ds(slot, 16)])
    plsc.subcore_barrier()
    pltpu.sync_copy(sharedF.at[pl.ds(pslot, 16)], xf)
    ptotv = xf[...]

    hv = jnp.full((16,), h, jnp.int32)
    T0v = jnp.where(hv == 0, mytotv, ptotv)
    T1v = jnp.where(hv == 0, ptotv, mytotv)
    ridx = jnp.minimum(r0 + iota, B - 1)
    ruv = plsc.load_gather(ru_v, [ridx])
    threshv = ruv * (T0v + T1v)
    c0v = T0v >= threshv
    bstartv = jnp.where(hv == 0, jnp.zeros((16,), jnp.float32), T0v)

    gidxs, bases = [], []
    for i in range(8):
        t_r = threshv[i]
        b0 = bstartv[i]

        def p2b(j, carry, i=i, t_r=t_r):
            bb, cnt = carry
            v = subsums[i, pl.ds(j * 16, 16)]
            cum = bb + plsc.cumsum(v)
            cumbuf[i, pl.ds(j * 16, 16)] = cum
            cnt = cnt + plsc.all_reduce_population_count(cum < t_r)
            return bb + jnp.sum(v), cnt

        _, cntv = lax.fori_loop(0, SPAD // 16, p2b,
                                (b0, jnp.zeros((16,), jnp.int32)))
        gidx = jnp.minimum(jnp.max(cntv), SPH - 1)
        base = jnp.where(gidx > 0,
                         _get2(cumbuf, i, jnp.maximum(gidx - 1, 0)), b0)
        gidxs.append(gidx)
        bases.append(base)

    recv = jnp.zeros((16,), jnp.int32)
    for wave in range(2):
        descs = []
        for j in range(4):
            i = 4 * wave + j
            offrel = pl.multiple_of(
                jnp.minimum(SUBW * gidxs[i], NFULL * CW), 128)
            off = pl.multiple_of(hoff + offrel, 128)
            bi, seg = j // 2, SUBW * (j % 2)
            for src, dstb in ((d_hbm, dblks[bi]), (t_hbm, tblks[bi])):
                c = pltpu.async_copy(
                    src.at[pl.ds(r0, 8), pl.ds(off, SUBW)],
                    dstb.at[:, pl.ds(seg, SUBW)], sems[0])
                descs.append(c)
        for c in descs:
            c.wait()

        for j in range(4):
            i = 4 * wave + j
            bi, seg = j // 2, SUBW * (j % 2)
            t_r = threshv[i]
            rowv = jnp.full((16,), i, jnp.int32)
            zi = jnp.zeros((16,), jnp.int32)

            def p3_main(bi=bi, seg=seg, t_r=t_r, rowv=rowv, base=bases[i]):
                def stepm(g, carry):
                    bb, cv = carry
                    colv = seg + g * 16 + iota
                    dd = plsc.load_gather(dblks[bi], [rowv, colv])
                    tt = plsc.load_gather(tblks[bi], [rowv, colv])
                    res = jnp.maximum(tt - dd, jnp.float32(0.0))
                    cum = plsc.cumsum(res)
                    cv = cv + plsc.all_reduce_population_count(
                        (bb + cum) < t_r)
                    return bb + jnp.sum(res), cv

                return lax.fori_loop(0, GP3, stepm, (base, zi))[1]

            def p3_tail(dref, tref, ngrp, t_r=t_r, rowv=rowv,
                        base=bases[i]):
                def stept(g, carry):
                    bb, cv = carry
                    colv = g * 16 + iota
                    dd = plsc.load_gather(dref, [rowv, colv])
                    tt = plsc.load_gather(tref, [rowv, colv])
                    res = jnp.maximum(tt - dd, jnp.float32(0.0))
                    cum = plsc.cumsum(res)
                    cv = cv + plsc.all_reduce_population_count(
                        (bb + cum) < t_r)
                    return bb + jnp.sum(res), cv

                return lax.fori_loop(0, ngrp, stept, (base, zi))[1]

            cv = lax.cond(
                gidxs[i] == SPH - 1,
                lambda: lax.cond(h == 1,
                                 lambda: p3_tail(dtl32, ttl32, 2),
                                 lambda: p3_tail(dtl, ttl, 8)),
                p3_main)
            rec = jnp.minimum(hoff + SUBW * gidxs[i] + jnp.max(cv), V - 1)
            recv = jnp.where(iota == i, jnp.full((16,), rec), recv)

    xi[...] = recv
    pltpu.sync_copy(xi, sharedI.at[pl.ds(slot, 16)])
    plsc.subcore_barrier()

    @pl.when(h == 0)
    def _():
        pltpu.sync_copy(ids_hbm, ids_v)
        pltpu.sync_copy(bon_hbm, bon_v)
        pltpu.sync_copy(u_hbm, u_v)
        pltpu.sync_copy(sharedI.at[pl.ds(pslot, 16)], xi)
        prec = xi[...]
        rec_final = jnp.where(c0v, recv, prec)

        outv = jnp.zeros((16,), jnp.int32)
        for i in range(8):
            tid_i = _get1(ids_v, r0 + i)
            intail = tid_i >= 99968
            walign = pl.multiple_of((tid_i // 128) * 128, 128)
            seg = 128 * i

            @pl.when(jnp.logical_not(intail))
            def _(walign=walign, seg=seg):
                pltpu.sync_copy(
                    d_hbm.at[pl.ds(r0, 8), pl.ds(walign, 128)],
                    dblks[1].at[:, pl.ds(seg, 128)])
                pltpu.sync_copy(
                    t_hbm.at[pl.ds(r0, 8), pl.ds(walign, 128)],
                    tblks[1].at[:, pl.ds(seg, 128)])

            @pl.when(intail)
            def _():
                pltpu.sync_copy(
                    d_hbm.at[pl.ds(r0, 8), pl.ds(99968, 32)], qd32)
                pltpu.sync_copy(
                    t_hbm.at[pl.ds(r0, 8), pl.ds(99968, 32)], qt32)

            rowv = jnp.full((16,), i, jnp.int32)

            def qp_main(rowv=rowv, seg=seg, tid_i=tid_i, walign=walign):
                locv = jnp.full((16,), seg + (tid_i - walign), jnp.int32)
                return (plsc.load_gather(dblks[1], [rowv, locv])[0],
                        plsc.load_gather(tblks[1], [rowv, locv])[0])

            def qp_tail(rowv=rowv, tid_i=tid_i):
                locv = jnp.full((16,), tid_i - 99968, jnp.int32)
                return (plsc.load_gather(qd32, [rowv, locv])[0],
                        plsc.load_gather(qt32, [rowv, locv])[0])

            q, p = lax.cond(intail, qp_tail, qp_main)
            accept = (_get1(u_v, r0 + i) * q) < p
            out0 = jnp.where(accept, tid_i, rec_final[i])
            out1 = jnp.where(accept, _get1(bon_v, r0 + i), jnp.int32(-1))
            outv = jnp.where(iota == 2 * i, jnp.full((16,), out0), outv)
            outv = jnp.where(iota == 2 * i + 1, jnp.full((16,), out1), outv)

        outb[...] = outv
        pltpu.sync_copy(outb, out_hbm.at[pl.ds(pl.multiple_of(gt * 16, 8),
                                               16)])


@jax.jit
def _sc_sampler(draft_probs, target_probs, draft_token_ids,
                bonus_token_ids, uniform_samples, residual_uniform):
    mesh = plsc.VectorSubcoreMesh(core_axis_name="c", subcore_axis_name="s")
    return pl.kernel(
        _sc_body,
        out_type=jax.ShapeDtypeStruct((B * 2,), jnp.int32),
        mesh=mesh,
        compiler_params=pltpu.CompilerParams(use_tc_tiling_on_sc=True,
                                             needs_layout_passes=False),
        scratch_types=[
            [pltpu.VMEM((8, CW), jnp.float32) for _ in range(2)],
            [pltpu.VMEM((8, CW), jnp.float32) for _ in range(2)],
            pltpu.VMEM((8, 128), jnp.float32),
            pltpu.VMEM((8, 128), jnp.float32),
            pltpu.VMEM((8, 32), jnp.float32),
            pltpu.VMEM((8, 32), jnp.float32),
            pltpu.VMEM((8, 32), jnp.float32),
            pltpu.VMEM((8, 32), jnp.float32),
            pltpu.VMEM((8, SPAD), jnp.float32),
            pltpu.VMEM((8, SPAD), jnp.float32),
            pltpu.VMEM((B,), jnp.int32),
            pltpu.VMEM((B,), jnp.int32),
            pltpu.VMEM((B,), jnp.float32),
            pltpu.VMEM((B,), jnp.float32),
            pltpu.VMEM((16,), jnp.float32),
            pltpu.VMEM((16,), jnp.int32),
            pltpu.VMEM_SHARED((256,), jnp.float32),
            pltpu.VMEM_SHARED((256,), jnp.int32),
            pltpu.VMEM((16,), jnp.int32),
            [pltpu.SemaphoreType.DMA for _ in range(2)],
        ],
    )(draft_probs, target_probs, draft_token_ids, bonus_token_ids,
      uniform_samples, residual_uniform)


def kernel(draft_probs, target_probs, draft_token_ids, bonus_token_ids,
           num_draft_tokens, uniform_samples, residual_uniform):
    del num_draft_tokens
    flat = _sc_sampler(draft_probs, target_probs, draft_token_ids,
                       bonus_token_ids, uniform_samples, residual_uniform)
    output_token_ids = flat.reshape(B, 2)
    accept = (output_token_ids[:, 1] != jnp.int32(-1)).astype(jnp.int32)
    num_accepted = accept + 1
    recovered_counts = 1 - accept
    return (output_token_ids, num_accepted, accept, recovered_counts, accept)


# speedup vs baseline: 1.0585x; 1.0585x over previous
import jax
import jax.numpy as jnp
from jax import lax
from jax.experimental import pallas as pl
from jax.experimental.pallas import tpu as pltpu
from jax.experimental.pallas import tpu_sc as plsc

B = 128
V = 100000
HW = 50048
CW = 2560
NFULL = 19
SUBW = 1280
SPH = 40
SPAD = 48
GP3 = SUBW // 16

_info = plsc.get_sparse_core_info()
NC, NS = _info.num_cores, _info.num_subcores


def _iota16():
    return lax.iota(jnp.int32, 16)


def _get1(ref, idx):
    return plsc.load_gather(ref, [jnp.full((16,), idx, jnp.int32)])[0]


def _get2(ref, i, j):
    return plsc.load_gather(ref, [jnp.full((16,), i, jnp.int32),
                                  jnp.full((16,), j, jnp.int32)])[0]


def _set2(ref, i, j, val):
    plsc.store_scatter(ref, [jnp.full((16,), i, jnp.int32),
                             jnp.full((16,), j, jnp.int32)],
                       jnp.full((16,), val), mask=_iota16() == 0)


def _sc_body(d_hbm, t_hbm, ids_hbm, bon_hbm, u_hbm, ru_hbm, out_hbm,
             dblks, tblks, dtl, ttl, dtl32, ttl32, qd32, qt32,
             subsums, cumbuf,
             ids_v, bon_v, u_v, ru_v, xf, xi, sharedF, sharedI,
             outb, sems):
    cid = lax.axis_index("c")
    sid = lax.axis_index("s")
    gt = 8 * cid + sid // 2
    h = sid % 2
    r0 = pl.multiple_of(8 * gt, 8)
    hoff = HW * h
    iota = _iota16()

    pltpu.sync_copy(ru_hbm, ru_v)

    chunks = [(CW * k, CW, 2) for k in range(NFULL)] + [(CW * NFULL, 1280, 1)]

    def startch(k):
        off, w, _ = chunks[k]
        b = k % 2
        col = pl.multiple_of(hoff + off, 128)
        pltpu.async_copy(d_hbm.at[pl.ds(r0, 8), pl.ds(col, w)],
                         dblks[b].at[:, pl.ds(0, w)], sems[b])
        pltpu.async_copy(t_hbm.at[pl.ds(r0, 8), pl.ds(col, w)],
                         tblks[b].at[:, pl.ds(0, w)], sems[b])

    def waitch(k):
        off, w, _ = chunks[k]
        b = k % 2
        col = pl.multiple_of(hoff + off, 128)
        pltpu.make_async_copy(d_hbm.at[pl.ds(r0, 8), pl.ds(col, w)],
                              dblks[b].at[:, pl.ds(0, w)], sems[b]).wait()
        pltpu.make_async_copy(t_hbm.at[pl.ds(r0, 8), pl.ds(col, w)],
                              tblks[b].at[:, pl.ds(0, w)], sems[b]).wait()

    for i in range(8):
        plsc.store_scatter(subsums,
                           [jnp.full((16,), i, jnp.int32), SPH + iota],
                           jnp.zeros((16,), jnp.float32),
                           mask=iota < (SPAD - SPH))

    startch(0)
    for k, (off, w, nsub) in enumerate(chunks):
        if k + 1 < len(chunks):
            startch(k + 1)
        waitch(k)
        db, tb = dblks[k % 2], tblks[k % 2]

        def row_body(row, _, db=db, tb=tb, k=k):
            rowv = jnp.full((16,), row, jnp.int32)

            def sub_body(s, _2):
                def grp(g, carry):
                    col = carry[0]
                    accs = list(carry[1:])
                    for u in range(8):
                        cu = col + u * 16
                        dd = db[row, pl.ds(cu, 16)]
                        tt = tb[row, pl.ds(cu, 16)]
                        accs[u % 4] = accs[u % 4] + jnp.maximum(
                            tt - dd, jnp.float32(0.0))
                    return (col + 128,) + tuple(accs)

                z = jnp.zeros((16,), jnp.float32)
                col0 = s * SUBW
                a = lax.fori_loop(0, GP3 // 8, grp, (col0, z, z, z, z))
                tot = jnp.sum((a[1] + a[2]) + (a[3] + a[4]))
                plsc.store_scatter(
                    subsums,
                    [rowv, jnp.full((16,), 2 * k, jnp.int32) + s],
                    jnp.full((16,), tot), mask=iota == 0)
                return _2

            return lax.fori_loop(0, nsub, sub_body, _)

        lax.fori_loop(0, 8, row_body, jnp.int32(0))

    @pl.when(h == 0)
    def _():
        pltpu.sync_copy(d_hbm.at[pl.ds(r0, 8), pl.ds(49920, 128)], dtl)
        pltpu.sync_copy(t_hbm.at[pl.ds(r0, 8), pl.ds(49920, 128)], ttl)

    @pl.when(h == 1)
    def _():
        pltpu.sync_copy(d_hbm.at[pl.ds(r0, 8), pl.ds(99968, 32)], dtl32)
        pltpu.sync_copy(t_hbm.at[pl.ds(r0, 8), pl.ds(99968, 32)], ttl32)

    def _tail_accum(dref, tref, ngrp):
        def tail_body(row, _):
            rowv = jnp.full((16,), row, jnp.int32)

            def tgrp(g, acc):
                colv = g * 16 + iota
                dd = plsc.load_gather(dref, [rowv, colv])
                tt = plsc.load_gather(tref, [rowv, colv])
                return acc + jnp.maximum(tt - dd, jnp.float32(0.0))

            acc = lax.fori_loop(0, ngrp, tgrp,
                                jnp.zeros((16,), jnp.float32))
            plsc.store_scatter(subsums,
                               [rowv, jnp.full((16,), SPH - 1, jnp.int32)],
                               jnp.full((16,), jnp.sum(acc)),
                               mask=iota == 0)
            return _

        lax.fori_loop(0, 8, tail_body, jnp.int32(0))

    @pl.when(h == 0)
    def _():
        _tail_accum(dtl, ttl, 8)

    @pl.when(h == 1)
    def _():
        _tail_accum(dtl32, ttl32, 2)

    mytotv = jnp.zeros((16,), jnp.float32)
    for i in range(8):
        acc = jnp.zeros((16,), jnp.float32)

        def p2a(j, acc, i=i):
            return acc + subsums[i, pl.ds(j * 16, 16)]

        acc = lax.fori_loop(0, SPAD // 16, p2a, acc)
        mytotv = jnp.where(iota == i, jnp.full((16,), jnp.sum(acc)), mytotv)

    slot = pl.multiple_of(sid * 16, 8)
    pslot = pl.multiple_of((sid ^ 1) * 16, 8)
    xf[...] = mytotv
    pltpu.sync_copy(xf, sharedF.at[pl.ds(slot, 16)])
    plsc.subcore_barrier()
    pltpu.sync_copy(sharedF.at[pl.ds(pslot, 16)], xf)
    ptotv = xf[...]

    hv = jnp.full((16,), h, jnp.int32)
    T0v = jnp.where(hv == 0, mytotv, ptotv)
    T1v = jnp.where(hv == 0, ptotv, mytotv)
    ridx = jnp.minimum(r0 + iota, B - 1)
    ruv = plsc.load_gather(ru_v, [ridx])
    threshv = ruv * (T0v + T1v)
    c0v = T0v >= threshv
    bstartv = jnp.where(hv == 0, jnp.zeros((16,), jnp.float32), T0v)

    gidxs, bases = [], []
    for i in range(8):
        t_r = threshv[i]
        b0 = bstartv[i]

        def p2b(j, carry, i=i, t_r=t_r):
            bb, cnt = carry
            v = subsums[i, pl.ds(j * 16, 16)]
            cum = bb + plsc.cumsum(v)
            cumbuf[i, pl.ds(j * 16, 16)] = cum
            cnt = cnt + plsc.all_reduce_population_count(cum < t_r)
            return bb + jnp.sum(v), cnt

        _, cntv = lax.fori_loop(0, SPAD // 16, p2b,
                                (b0, jnp.zeros((16,), jnp.int32)))
        gidx = jnp.minimum(jnp.max(cntv), SPH - 1)
        base = jnp.where(gidx > 0,
                         _get2(cumbuf, i, jnp.maximum(gidx - 1, 0)), b0)
        gidxs.append(gidx)
        bases.append(base)

    recv = jnp.zeros((16,), jnp.int32)
    for wave in range(2):
        descs = []
        for j in range(4):
            i = 4 * wave + j
            offrel = pl.multiple_of(
                jnp.minimum(SUBW * gidxs[i], NFULL * CW), 128)
            off = pl.multiple_of(hoff + offrel, 128)
            bi, seg = j // 2, SUBW * (j % 2)
            for src, dstb in ((d_hbm, dblks[bi]), (t_hbm, tblks[bi])):
                c = pltpu.async_copy(
                    src.at[pl.ds(r0, 8), pl.ds(off, SUBW)],
                    dstb.at[:, pl.ds(seg, SUBW)], sems[0])
                descs.append(c)
        for c in descs:
            c.wait()

        for j in range(4):
            i = 4 * wave + j
            bi, seg = j // 2, SUBW * (j % 2)
            t_r = threshv[i]
            rowv = jnp.full((16,), i, jnp.int32)
            zi = jnp.zeros((16,), jnp.int32)

            def p3_main(bi=bi, seg=seg, t_r=t_r, rowv=rowv, base=bases[i]):
                def stepm(g, carry):
                    bb, cv = carry
                    colv = seg + g * 16 + iota
                    dd = plsc.load_gather(dblks[bi], [rowv, colv])
                    tt = plsc.load_gather(tblks[bi], [rowv, colv])
                    res = jnp.maximum(tt - dd, jnp.float32(0.0))
                    cum = plsc.cumsum(res)
                    cv = cv + plsc.all_reduce_population_count(
                        (bb + cum) < t_r)
                    return bb + jnp.sum(res), cv

                return lax.fori_loop(0, GP3, stepm, (base, zi))[1]

            def p3_tail(dref, tref, ngrp, t_r=t_r, rowv=rowv,
                        base=bases[i]):
                def stept(g, carry):
                    bb, cv = carry
                    colv = g * 16 + iota
                    dd = plsc.load_gather(dref, [rowv, colv])
                    tt = plsc.load_gather(tref, [rowv, colv])
                    res = jnp.maximum(tt - dd, jnp.float32(0.0))
                    cum = plsc.cumsum(res)
                    cv = cv + plsc.all_reduce_population_count(
                        (bb + cum) < t_r)
                    return bb + jnp.sum(res), cv

                return lax.fori_loop(0, ngrp, stept, (base, zi))[1]

            cv = lax.cond(
                gidxs[i] == SPH - 1,
                lambda: lax.cond(h == 1,
                                 lambda: p3_tail(dtl32, ttl32, 2),
                                 lambda: p3_tail(dtl, ttl, 8)),
                p3_main)
            rec = jnp.minimum(hoff + SUBW * gidxs[i] + jnp.max(cv), V - 1)
            recv = jnp.where(iota == i, jnp.full((16,), rec), recv)

    xi[...] = recv
    pltpu.sync_copy(xi, sharedI.at[pl.ds(slot, 16)])
    plsc.subcore_barrier()

    @pl.when(h == 0)
    def _():
        pltpu.sync_copy(ids_hbm, ids_v)
        pltpu.sync_copy(bon_hbm, bon_v)
        pltpu.sync_copy(u_hbm, u_v)
        pltpu.sync_copy(sharedI.at[pl.ds(pslot, 16)], xi)
        prec = xi[...]
        rec_final = jnp.where(c0v, recv, prec)

        outv = jnp.zeros((16,), jnp.int32)
        for i in range(8):
            tid_i = _get1(ids_v, r0 + i)
            intail = tid_i >= 99968
            walign = pl.multiple_of((tid_i // 128) * 128, 128)
            seg = 128 * i

            @pl.when(jnp.logical_not(intail))
            def _(walign=walign, seg=seg):
                pltpu.sync_copy(
                    d_hbm.at[pl.ds(r0, 8), pl.ds(walign, 128)],
                    dblks[1].at[:, pl.ds(seg, 128)])
                pltpu.sync_copy(
                    t_hbm.at[pl.ds(r0, 8), pl.ds(walign, 128)],
                    tblks[1].at[:, pl.ds(seg, 128)])

            @pl.when(intail)
            def _():
                pltpu.sync_copy(
                    d_hbm.at[pl.ds(r0, 8), pl.ds(99968, 32)], qd32)
                pltpu.sync_copy(
                    t_hbm.at[pl.ds(r0, 8), pl.ds(99968, 32)], qt32)

            rowv = jnp.full((16,), i, jnp.int32)

            def qp_main(rowv=rowv, seg=seg, tid_i=tid_i, walign=walign):
                locv = jnp.full((16,), seg + (tid_i - walign), jnp.int32)
                return (plsc.load_gather(dblks[1], [rowv, locv])[0],
                        plsc.load_gather(tblks[1], [rowv, locv])[0])

            def qp_tail(rowv=rowv, tid_i=tid_i):
                locv = jnp.full((16,), tid_i - 99968, jnp.int32)
                return (plsc.load_gather(qd32, [rowv, locv])[0],
                        plsc.load_gather(qt32, [rowv, locv])[0])

            q, p = lax.cond(intail, qp_tail, qp_main)
            accept = (_get1(u_v, r0 + i) * q) < p
            out0 = jnp.where(accept, tid_i, rec_final[i])
            out1 = jnp.where(accept, _get1(bon_v, r0 + i), jnp.int32(-1))
            outv = jnp.where(iota == 2 * i, jnp.full((16,), out0), outv)
            outv = jnp.where(iota == 2 * i + 1, jnp.full((16,), out1), outv)

        outb[...] = outv
        pltpu.sync_copy(outb, out_hbm.at[pl.ds(pl.multiple_of(gt * 16, 8),
                                               16)])


@jax.jit
def _sc_sampler(draft_probs, target_probs, draft_token_ids,
                bonus_token_ids, uniform_samples, residual_uniform):
    mesh = plsc.VectorSubcoreMesh(core_axis_name="c", subcore_axis_name="s")
    return pl.kernel(
        _sc_body,
        out_type=jax.ShapeDtypeStruct((B * 2,), jnp.int32),
        mesh=mesh,
        compiler_params=pltpu.CompilerParams(use_tc_tiling_on_sc=True,
                                             needs_layout_passes=False),
        scratch_types=[
            [pltpu.VMEM((8, CW), jnp.float32) for _ in range(2)],
            [pltpu.VMEM((8, CW), jnp.float32) for _ in range(2)],
            pltpu.VMEM((8, 128), jnp.float32),
            pltpu.VMEM((8, 128), jnp.float32),
            pltpu.VMEM((8, 32), jnp.float32),
            pltpu.VMEM((8, 32), jnp.float32),
            pltpu.VMEM((8, 32), jnp.float32),
            pltpu.VMEM((8, 32), jnp.float32),
            pltpu.VMEM((8, SPAD), jnp.float32),
            pltpu.VMEM((8, SPAD), jnp.float32),
            pltpu.VMEM((B,), jnp.int32),
            pltpu.VMEM((B,), jnp.int32),
            pltpu.VMEM((B,), jnp.float32),
            pltpu.VMEM((B,), jnp.float32),
            pltpu.VMEM((16,), jnp.float32),
            pltpu.VMEM((16,), jnp.int32),
            pltpu.VMEM_SHARED((256,), jnp.float32),
            pltpu.VMEM_SHARED((256,), jnp.int32),
            pltpu.VMEM((16,), jnp.int32),
            [pltpu.SemaphoreType.DMA for _ in range(2)],
        ],
    )(draft_probs, target_probs, draft_token_ids, bonus_token_ids,
      uniform_samples, residual_uniform)


def kernel(draft_probs, target_probs, draft_token_ids, bonus_token_ids,
           num_draft_tokens, uniform_samples, residual_uniform):
    del num_draft_tokens
    flat = _sc_sampler(draft_probs, target_probs, draft_token_ids,
                       bonus_token_ids, uniform_samples, residual_uniform)
    output_token_ids = flat.reshape(B, 2)
    accept = (output_token_ids[:, 1] != jnp.int32(-1)).astype(jnp.int32)
    num_accepted = accept + 1
    recovered_counts = 1 - accept
    return (output_token_ids, num_accepted, accept, recovered_counts, accept)


# speedup vs baseline: 1.1125x; 1.0510x over previous
import jax
import jax.numpy as jnp
from jax import lax
from jax.experimental import pallas as pl
from jax.experimental.pallas import tpu as pltpu
from jax.experimental.pallas import tpu_sc as plsc

B = 128
V = 100000
HW = 50048
CW = 2560
NFULL = 19
SUBW = 1280
SPH = 40
SPAD = 48
GP3 = SUBW // 16

_info = plsc.get_sparse_core_info()
NC, NS = _info.num_cores, _info.num_subcores


def _iota16():
    return lax.iota(jnp.int32, 16)


def _get1(ref, idx):
    return plsc.load_gather(ref, [jnp.full((16,), idx, jnp.int32)])[0]


def _get2(ref, i, j):
    return plsc.load_gather(ref, [jnp.full((16,), i, jnp.int32),
                                  jnp.full((16,), j, jnp.int32)])[0]


def _set2(ref, i, j, val):
    plsc.store_scatter(ref, [jnp.full((16,), i, jnp.int32),
                             jnp.full((16,), j, jnp.int32)],
                       jnp.full((16,), val), mask=_iota16() == 0)


def _sc_body(d_hbm, t_hbm, ids_hbm, bon_hbm, u_hbm, ru_hbm, out_hbm,
             dblks, tblks, dtl, ttl, dtl32, ttl32, qd32, qt32,
             subsums, cumbuf,
             ids_v, bon_v, u_v, ru_v, xf, xi, sharedF, sharedI,
             outb, sems):
    cid = lax.axis_index("c")
    sid = lax.axis_index("s")
    gt = 8 * cid + sid // 2
    h = sid % 2
    r0 = pl.multiple_of(8 * gt, 8)
    hoff = HW * h
    iota = _iota16()

    pltpu.sync_copy(ru_hbm, ru_v)

    chunks = [(CW * k, CW, 2) for k in range(NFULL)] + [(CW * NFULL, 1280, 1)]

    def startch(k):
        off, w, _ = chunks[k]
        b = k % 2
        col = pl.multiple_of(hoff + off, 128)
        pltpu.async_copy(d_hbm.at[pl.ds(r0, 8), pl.ds(col, w)],
                         dblks[b].at[:, pl.ds(0, w)], sems[b])
        pltpu.async_copy(t_hbm.at[pl.ds(r0, 8), pl.ds(col, w)],
                         tblks[b].at[:, pl.ds(0, w)], sems[b])

    def waitch(k):
        off, w, _ = chunks[k]
        b = k % 2
        col = pl.multiple_of(hoff + off, 128)
        pltpu.make_async_copy(d_hbm.at[pl.ds(r0, 8), pl.ds(col, w)],
                              dblks[b].at[:, pl.ds(0, w)], sems[b]).wait()
        pltpu.make_async_copy(t_hbm.at[pl.ds(r0, 8), pl.ds(col, w)],
                              tblks[b].at[:, pl.ds(0, w)], sems[b]).wait()

    for i in range(8):
        plsc.store_scatter(subsums,
                           [jnp.full((16,), i, jnp.int32), SPH + iota],
                           jnp.zeros((16,), jnp.float32),
                           mask=iota < (SPAD - SPH))

    startch(0)
    for k, (off, w, nsub) in enumerate(chunks):
        if k + 1 < len(chunks):
            startch(k + 1)
        waitch(k)
        db, tb = dblks[k % 2], tblks[k % 2]

        def row_body(row, _, db=db, tb=tb, k=k):
            rowv = jnp.full((16,), row, jnp.int32)

            def sub_body(s, _2):
                def grp(g, carry):
                    col = carry[0]
                    accs = list(carry[1:])
                    for u in range(8):
                        cu = col + u * 16
                        dd = db[row, pl.ds(cu, 16)]
                        tt = tb[row, pl.ds(cu, 16)]
                        accs[u % 4] = accs[u % 4] + jnp.maximum(
                            tt - dd, jnp.float32(0.0))
                    return (col + 128,) + tuple(accs)

                z = jnp.zeros((16,), jnp.float32)
                col0 = s * SUBW
                a = lax.fori_loop(0, GP3 // 8, grp, (col0, z, z, z, z))
                tot = jnp.sum((a[1] + a[2]) + (a[3] + a[4]))
                plsc.store_scatter(
                    subsums,
                    [rowv, jnp.full((16,), 2 * k, jnp.int32) + s],
                    jnp.full((16,), tot), mask=iota == 0)
                return _2

            return lax.fori_loop(0, nsub, sub_body, _)

        lax.fori_loop(0, 8, row_body, jnp.int32(0))

    @pl.when(h == 0)
    def _():
        pltpu.sync_copy(d_hbm.at[pl.ds(r0, 8), pl.ds(49920, 128)], dtl)
        pltpu.sync_copy(t_hbm.at[pl.ds(r0, 8), pl.ds(49920, 128)], ttl)

    @pl.when(h == 1)
    def _():
        pltpu.sync_copy(d_hbm.at[pl.ds(r0, 8), pl.ds(99968, 32)], dtl32)
        pltpu.sync_copy(t_hbm.at[pl.ds(r0, 8), pl.ds(99968, 32)], ttl32)

    def _tail_accum(dref, tref, ngrp):
        def tail_body(row, _):
            rowv = jnp.full((16,), row, jnp.int32)

            def tgrp(g, acc):
                colv = g * 16 + iota
                dd = plsc.load_gather(dref, [rowv, colv])
                tt = plsc.load_gather(tref, [rowv, colv])
                return acc + jnp.maximum(tt - dd, jnp.float32(0.0))

            acc = lax.fori_loop(0, ngrp, tgrp,
                                jnp.zeros((16,), jnp.float32))
            plsc.store_scatter(subsums,
                               [rowv, jnp.full((16,), SPH - 1, jnp.int32)],
                               jnp.full((16,), jnp.sum(acc)),
                               mask=iota == 0)
            return _

        lax.fori_loop(0, 8, tail_body, jnp.int32(0))

    @pl.when(h == 0)
    def _():
        _tail_accum(dtl, ttl, 8)

    @pl.when(h == 1)
    def _():
        _tail_accum(dtl32, ttl32, 2)

    mytotv = jnp.zeros((16,), jnp.float32)
    for i in range(8):
        acc = jnp.zeros((16,), jnp.float32)

        def p2a(j, acc, i=i):
            return acc + subsums[i, pl.ds(j * 16, 16)]

        acc = lax.fori_loop(0, SPAD // 16, p2a, acc)
        mytotv = jnp.where(iota == i, jnp.full((16,), jnp.sum(acc)), mytotv)

    slot = pl.multiple_of(sid * 16, 8)
    pslot = pl.multiple_of((sid ^ 1) * 16, 8)
    xf[...] = mytotv
    pltpu.sync_copy(xf, sharedF.at[pl.ds(slot, 16)])
    plsc.subcore_barrier()
    pltpu.sync_copy(sharedF.at[pl.ds(pslot, 16)], xf)
    ptotv = xf[...]

    hv = jnp.full((16,), h, jnp.int32)
    T0v = jnp.where(hv == 0, mytotv, ptotv)
    T1v = jnp.where(hv == 0, ptotv, mytotv)
    ridx = jnp.minimum(r0 + iota, B - 1)
    ruv = plsc.load_gather(ru_v, [ridx])
    threshv = ruv * (T0v + T1v)
    c0v = T0v >= threshv
    bstartv = jnp.where(hv == 0, jnp.zeros((16,), jnp.float32), T0v)

    gidxs, bases = [], []
    for i in range(8):
        t_r = threshv[i]
        b0 = bstartv[i]

        def p2b(j, carry, i=i, t_r=t_r):
            bb, cnt = carry
            v = subsums[i, pl.ds(j * 16, 16)]
            cum = bb + plsc.cumsum(v)
            cumbuf[i, pl.ds(j * 16, 16)] = cum
            cnt = cnt + plsc.all_reduce_population_count(cum < t_r)
            return bb + jnp.sum(v), cnt

        _, cntv = lax.fori_loop(0, SPAD // 16, p2b,
                                (b0, jnp.zeros((16,), jnp.int32)))
        gidx = jnp.minimum(jnp.max(cntv), SPH - 1)
        base = jnp.where(gidx > 0,
                         _get2(cumbuf, i, jnp.maximum(gidx - 1, 0)), b0)
        gidxs.append(gidx)
        bases.append(base)

    recv = jnp.zeros((16,), jnp.int32)
    for wave in range(2):
        descs = []
        for j in range(4):
            i = 4 * wave + j
            offrel = pl.multiple_of(
                jnp.minimum(SUBW * gidxs[i], NFULL * CW), 128)
            off = pl.multiple_of(hoff + offrel, 128)
            bi, seg = j // 2, SUBW * (j % 2)
            for src, dstb in ((d_hbm, dblks[bi]), (t_hbm, tblks[bi])):
                c = pltpu.async_copy(
                    src.at[pl.ds(r0, 8), pl.ds(off, SUBW)],
                    dstb.at[:, pl.ds(seg, SUBW)], sems[0])
                descs.append(c)
        for c in descs:
            c.wait()

        for j in range(4):
            i = 4 * wave + j
            bi, seg = j // 2, SUBW * (j % 2)
            t_r = threshv[i]
            rowv = jnp.full((16,), i, jnp.int32)
            zi = jnp.zeros((16,), jnp.int32)

            def p3_main(bi=bi, seg=seg, t_r=t_r, rowv=rowv, base=bases[i]):
                def stepm(g, carry):
                    bb, cv = carry
                    colv = seg + g * 16 + iota
                    dd = plsc.load_gather(dblks[bi], [rowv, colv])
                    tt = plsc.load_gather(tblks[bi], [rowv, colv])
                    res = jnp.maximum(tt - dd, jnp.float32(0.0))
                    cum = plsc.cumsum(res)
                    cv = cv + plsc.all_reduce_population_count(
                        (bb + cum) < t_r)
                    return bb + jnp.sum(res), cv

                return lax.fori_loop(0, GP3, stepm, (base, zi))[1]

            def p3_tail(dref, tref, ngrp, t_r=t_r, rowv=rowv,
                        base=bases[i]):
                def stept(g, carry):
                    bb, cv = carry
                    colv = g * 16 + iota
                    dd = plsc.load_gather(dref, [rowv, colv])
                    tt = plsc.load_gather(tref, [rowv, colv])
                    res = jnp.maximum(tt - dd, jnp.float32(0.0))
                    cum = plsc.cumsum(res)
                    cv = cv + plsc.all_reduce_population_count(
                        (bb + cum) < t_r)
                    return bb + jnp.sum(res), cv

                return lax.fori_loop(0, ngrp, stept, (base, zi))[1]

            cv = lax.cond(
                gidxs[i] == SPH - 1,
                lambda: lax.cond(h == 1,
                                 lambda: p3_tail(dtl32, ttl32, 2),
                                 lambda: p3_tail(dtl, ttl, 8)),
                p3_main)
            rec = jnp.minimum(hoff + SUBW * gidxs[i] + jnp.max(cv), V - 1)
            recv = jnp.where(iota == i, jnp.full((16,), rec), recv)

    xi[...] = recv
    pltpu.sync_copy(xi, sharedI.at[pl.ds(slot, 16)])
    plsc.subcore_barrier()

    @pl.when(h == 0)
    def _():
        pltpu.sync_copy(ids_hbm, ids_v)
        pltpu.sync_copy(bon_hbm, bon_v)
        pltpu.sync_copy(u_hbm, u_v)
        pltpu.sync_copy(sharedI.at[pl.ds(pslot, 16)], xi)
        prec = xi[...]
        rec_final = jnp.where(c0v, recv, prec)

        qdescs = []
        tids, waligns = [], []
        for i in range(8):
            tid_i = _get1(ids_v, r0 + i)
            walign = pl.multiple_of(
                jnp.minimum((tid_i // 128) * 128, V - 160), 128)
            tids.append(tid_i)
            waligns.append(walign)
            seg = 128 * i
            for src, dstb in ((d_hbm, dblks[1]), (t_hbm, tblks[1])):
                qdescs.append(pltpu.async_copy(
                    src.at[pl.ds(r0, 8), pl.ds(walign, 128)],
                    dstb.at[:, pl.ds(seg, 128)], sems[1]))
        qdescs.append(pltpu.async_copy(
            d_hbm.at[pl.ds(r0, 8), pl.ds(99968, 32)], qd32, sems[1]))
        qdescs.append(pltpu.async_copy(
            t_hbm.at[pl.ds(r0, 8), pl.ds(99968, 32)], qt32, sems[1]))
        for c in qdescs:
            c.wait()

        outv = jnp.zeros((16,), jnp.int32)
        for i in range(8):
            tid_i = tids[i]
            walign = waligns[i]
            intail = tid_i >= 99968
            seg = 128 * i
            rowv = jnp.full((16,), i, jnp.int32)

            def qp_main(rowv=rowv, seg=seg, tid_i=tid_i, walign=walign):
                locv = jnp.full((16,), seg + (tid_i - walign), jnp.int32)
                return (plsc.load_gather(dblks[1], [rowv, locv])[0],
                        plsc.load_gather(tblks[1], [rowv, locv])[0])

            def qp_tail(rowv=rowv, tid_i=tid_i):
                locv = jnp.full((16,), tid_i - 99968, jnp.int32)
                return (plsc.load_gather(qd32, [rowv, locv])[0],
                        plsc.load_gather(qt32, [rowv, locv])[0])

            q, p = lax.cond(intail, qp_tail, qp_main)
            accept = (_get1(u_v, r0 + i) * q) < p
            out0 = jnp.where(accept, tid_i, rec_final[i])
            out1 = jnp.where(accept, _get1(bon_v, r0 + i), jnp.int32(-1))
            outv = jnp.where(iota == 2 * i, jnp.full((16,), out0), outv)
            outv = jnp.where(iota == 2 * i + 1, jnp.full((16,), out1), outv)

        outb[...] = outv
        pltpu.sync_copy(outb, out_hbm.at[pl.ds(pl.multiple_of(gt * 16, 8),
                                               16)])


@jax.jit
def _sc_sampler(draft_probs, target_probs, draft_token_ids,
                bonus_token_ids, uniform_samples, residual_uniform):
    mesh = plsc.VectorSubcoreMesh(core_axis_name="c", subcore_axis_name="s")
    return pl.kernel(
        _sc_body,
        out_type=jax.ShapeDtypeStruct((B * 2,), jnp.int32),
        mesh=mesh,
        compiler_params=pltpu.CompilerParams(use_tc_tiling_on_sc=True,
                                             needs_layout_passes=False),
        scratch_types=[
            [pltpu.VMEM((8, CW), jnp.float32) for _ in range(2)],
            [pltpu.VMEM((8, CW), jnp.float32) for _ in range(2)],
            pltpu.VMEM((8, 128), jnp.float32),
            pltpu.VMEM((8, 128), jnp.float32),
            pltpu.VMEM((8, 32), jnp.float32),
            pltpu.VMEM((8, 32), jnp.float32),
            pltpu.VMEM((8, 32), jnp.float32),
            pltpu.VMEM((8, 32), jnp.float32),
            pltpu.VMEM((8, SPAD), jnp.float32),
            pltpu.VMEM((8, SPAD), jnp.float32),
            pltpu.VMEM((B,), jnp.int32),
            pltpu.VMEM((B,), jnp.int32),
            pltpu.VMEM((B,), jnp.float32),
            pltpu.VMEM((B,), jnp.float32),
            pltpu.VMEM((16,), jnp.float32),
            pltpu.VMEM((16,), jnp.int32),
            pltpu.VMEM_SHARED((256,), jnp.float32),
            pltpu.VMEM_SHARED((256,), jnp.int32),
            pltpu.VMEM((16,), jnp.int32),
            [pltpu.SemaphoreType.DMA for _ in range(2)],
        ],
    )(draft_probs, target_probs, draft_token_ids, bonus_token_ids,
      uniform_samples, residual_uniform)


def kernel(draft_probs, target_probs, draft_token_ids, bonus_token_ids,
           num_draft_tokens, uniform_samples, residual_uniform):
    del num_draft_tokens
    flat = _sc_sampler(draft_probs, target_probs, draft_token_ids,
                       bonus_token_ids, uniform_samples, residual_uniform)
    output_token_ids = flat.reshape(B, 2)
    accept = (output_token_ids[:, 1] != jnp.int32(-1)).astype(jnp.int32)
    num_accepted = accept + 1
    recovered_counts = 1 - accept
    return (output_token_ids, num_accepted, accept, recovered_counts, accept)
